# Initial kernel scaffold; baseline (speedup 1.0000x reference)
import jax
import jax.numpy as jnp
from jax.experimental import pallas as pl


def kernel(agts, ctx, agt_ctrs, ctx_ctrs, W_d0, b_d0, W_d1, g_d, be_d, W_q, g_q, be_q, W_c0, g_c0, be_c0, W_c1, W_a, g_n, be_n, W_l, g_l, be_l, hi, wi):
    raise NotImplementedError("write your pallas kernel here")


# speedup vs baseline: 3.8561x; 3.8561x over previous
import functools

import jax
import jax.numpy as jnp
from jax import lax
from jax.experimental import pallas as pl
from jax.experimental.pallas import tpu as pltpu
from jax.experimental.pallas import tpu_sc as plsc

_NC = 2
_NS = 16
_NW = _NC * _NS
_CH = 80
_NPAD = 10240

_EPS = 1e-5


def _gn(x, g, b):
    mu = jnp.mean(x, axis=1, keepdims=True)
    var = jnp.mean((x - mu) ** 2, axis=1, keepdims=True)
    return (x - mu) / jnp.sqrt(var + _EPS) * g + b


def _pre_body(agts_r, ctx_r, actr_r, cctr_r, wd0_r, bd0_r, wq_r, gq_r, beq_r,
              wc0q_r, wc0x_r, wa_r, agt_tab_r, ctx_tab_r, base_r):
    ac = actr_r[...]
    cc = cctr_r[...]
    wd0 = wd0_r[...]
    a = ac[:, 0:1] * wd0[0:1, :] + ac[:, 1:2] * wd0[1:2, :] + bd0_r[...]
    c = cc[:, 0:1] * wd0[0:1, :] + cc[:, 1:2] * wd0[1:2, :]
    agts = agts_r[...]
    q = jax.nn.relu(_gn(jnp.dot(agts, wq_r[...], preferred_element_type=jnp.float32),
                        gq_r[...], beq_r[...]))
    agt_tab_r[:, :128] = a
    agt_tab_r[:, 128:] = jnp.dot(q, wc0q_r[...], preferred_element_type=jnp.float32)
    ctx_tab_r[:, :128] = -c
    ctx_tab_r[:, 128:] = jnp.dot(ctx_r[...], wc0x_r[...], preferred_element_type=jnp.float32)
    base_r[...] = jnp.dot(agts, wa_r[...], preferred_element_type=jnp.float32)


def _build_tables(agts, ctx, agt_ctrs, ctx_ctrs, W_d0, b_d0, W_q, g_q, be_q,
                  Wc0_q, Wc0_x, W_a):
    n, d = agts.shape
    blk = 1000
    grid = n // blk
    full = lambda r, c: pl.BlockSpec((r, c), lambda i: (0, 0))
    return pl.pallas_call(
        _pre_body,
        grid=(grid,),
        in_specs=[
            pl.BlockSpec((blk, d), lambda i: (i, 0)),
            pl.BlockSpec((blk, d), lambda i: (i, 0)),
            pl.BlockSpec((blk, 2), lambda i: (i, 0)),
            pl.BlockSpec((blk, 2), lambda i: (i, 0)),
            full(2, d), full(1, d), full(d, d), full(1, d), full(1, d),
            full(d, d), full(d, d), full(d, d),
        ],
        out_specs=[
            pl.BlockSpec((blk, 2 * d), lambda i: (i, 0)),
            pl.BlockSpec((blk, 2 * d), lambda i: (i, 0)),
            pl.BlockSpec((blk, d), lambda i: (i, 0)),
        ],
        out_shape=[
            jax.ShapeDtypeStruct((n, 2 * d), jnp.float32),
            jax.ShapeDtypeStruct((n, 2 * d), jnp.float32),
            jax.ShapeDtypeStruct((n, d), jnp.float32),
        ],
    )(agts, ctx, agt_ctrs, ctx_ctrs, W_d0, b_d0, W_q, g_q, be_q,
      Wc0_q, Wc0_x, W_a)


def _gather_rows(agt_tab, ctx_tab, hi, wi):
    e = hi.shape[0]
    n, w = agt_tab.shape
    per_w = e // _NW
    nch = per_w // _CH
    mesh = plsc.VectorSubcoreMesh(core_axis_name="c", subcore_axis_name="s",
                                  num_cores=_NC, num_subcores=_NS)

    @functools.partial(
        pl.kernel,
        out_type=(jax.ShapeDtypeStruct((e, w), jnp.float32),
                  jax.ShapeDtypeStruct((e, w), jnp.float32)),
        mesh=mesh,
        scratch_types=[
            pltpu.VMEM((per_w,), jnp.int32),
            pltpu.VMEM((per_w,), jnp.int32),
            pltpu.VMEM((_CH, w), jnp.float32),
            pltpu.VMEM((_CH, w), jnp.float32),
            pltpu.SemaphoreType.DMA,
            pltpu.SemaphoreType.DMA,
        ],
    )
    def gather_k(agt_hbm, ctx_hbm, hi_hbm, wi_hbm, g1_hbm, g2_hbm,
                 hi_v, wi_v, r1, r2, s1, s2):
        wid = lax.axis_index("s") * _NC + lax.axis_index("c")
        e0 = wid * per_w
        pltpu.sync_copy(hi_hbm.at[pl.ds(e0, per_w)], hi_v)
        pltpu.sync_copy(wi_hbm.at[pl.ds(e0, per_w)], wi_v)

        def body(j, carry):
            base = e0 + j * _CH
            cp1 = pltpu.async_copy(agt_hbm.at[hi_v.at[pl.ds(j * _CH, _CH)]], r1, s1)
            cp2 = pltpu.async_copy(ctx_hbm.at[wi_v.at[pl.ds(j * _CH, _CH)]], r2, s2)
            cp1.wait()
            cp2.wait()
            pltpu.sync_copy(r1, g1_hbm.at[pl.ds(base, _CH)])
            pltpu.sync_copy(r2, g2_hbm.at[pl.ds(base, _CH)])
            return carry

        lax.fori_loop(0, nch, body, 0)

    return gather_k(agt_tab, ctx_tab, hi, wi)


def _edge_body(g1_r, g2_r, wd1_r, gd_r, bed_r, wc0d_r, gc0_r, bec0_r, h_r):
    g1 = g1_r[...]
    g2 = g2_r[...]
    d1 = jax.nn.relu(g1[:, :128] + g2[:, :128])
    d2 = jax.nn.relu(_gn(jnp.dot(d1, wd1_r[...], preferred_element_type=jnp.float32),
                         gd_r[...], bed_r[...]))
    pre = jnp.dot(d2, wc0d_r[...], preferred_element_type=jnp.float32)
    pre = pre + g1[:, 128:] + g2[:, 128:]
    h_r[...] = jax.nn.relu(_gn(pre, gc0_r[...], bec0_r[...]))


def _edge_mlp(g1, g2, W_d1, g_d, be_d, Wc0_d, g_c0, be_c0):
    e, w = g1.shape
    d = w // 2
    blk = 2000
    grid = e // blk
    full = lambda r, c: pl.BlockSpec((r, c), lambda i: (0, 0))
    return pl.pallas_call(
        _edge_body,
        grid=(grid,),
        in_specs=[
            pl.BlockSpec((blk, w), lambda i: (i, 0)),
            pl.BlockSpec((blk, w), lambda i: (i, 0)),
            full(d, d), full(1, d), full(1, d),
            full(d, d), full(1, d), full(1, d),
        ],
        out_specs=pl.BlockSpec((blk, d), lambda i: (i, 0)),
        out_shape=jax.ShapeDtypeStruct((e, d), jnp.float32),
    )(g1, g2, W_d1, g_d, be_d, Wc0_d, g_c0, be_c0)


def _scatter_add(h, hi):
    e, d = h.shape
    per_w = e // _NW
    nch = per_w // _CH
    stripe = _NPAD // _NS
    zrows = stripe // 5
    mesh = plsc.VectorSubcoreMesh(core_axis_name="c", subcore_axis_name="s",
                                  num_cores=_NC, num_subcores=_NS)

    @functools.partial(
        pl.kernel,
        out_type=jax.ShapeDtypeStruct((_NC, _NPAD, d), jnp.float32),
        mesh=mesh,
        scratch_types=[
            pltpu.VMEM((_CH,), jnp.int32),
            pltpu.VMEM((_CH, d), jnp.float32),
            pltpu.VMEM((zrows, d), jnp.float32),
            pltpu.VMEM_SHARED((_NPAD, d), jnp.float32),
        ],
    )
    def scatter_k(h_hbm, hi_hbm, s_out, hi_c, hbuf, zbuf, s_sh):
        cid = lax.axis_index("c")
        sid = lax.axis_index("s")
        wid = sid * _NC + cid
        e0 = wid * per_w

        def zb(i, carry):
            zbuf[i // 8, pl.ds((i % 8) * 16, 16)] = jnp.zeros((16,), jnp.float32)
            return carry

        lax.fori_loop(0, zrows * 8, zb, 0)

        def zc(p, carry):
            pltpu.sync_copy(zbuf, s_sh.at[pl.ds(sid * stripe + p * zrows, zrows)])
            return carry

        lax.fori_loop(0, 5, zc, 0)
        plsc.subcore_barrier()

        def body(j, carry):
            base = e0 + j * _CH
            pltpu.sync_copy(hi_hbm.at[pl.ds(base, _CH)], hi_c)
            pltpu.sync_copy(h_hbm.at[pl.ds(base, _CH)], hbuf)
            pltpu.sync_copy(hbuf, s_sh.at[hi_c], add=True)
            return carry

        lax.fori_loop(0, nch, body, 0)
        plsc.subcore_barrier()
        pltpu.sync_copy(s_sh.at[pl.ds(sid * stripe, stripe)],
                        s_out.at[cid, pl.ds(sid * stripe, stripe)])

    return scatter_k(h, hi)


def _final_body(s_r, base_r, agts_r, wc1_r, gn_r, ben_r, wl_r, gl_r, bel_r, o_r):
    s = s_r[0] + s_r[1]
    out = base_r[...] + jnp.dot(s, wc1_r[...], preferred_element_type=jnp.float32)
    out = jax.nn.relu(_gn(out, gn_r[...], ben_r[...]))
    out = _gn(jnp.dot(out, wl_r[...], preferred_element_type=jnp.float32),
              gl_r[...], bel_r[...])
    o_r[...] = jax.nn.relu(out + agts_r[...])


def _final(s_parts, base, agts, W_c1, g_n, be_n, W_l, g_l, be_l):
    n, d = agts.shape
    blk = 1000
    grid = n // blk
    full = lambda r, c: pl.BlockSpec((r, c), lambda i: (0, 0))
    return pl.pallas_call(
        _final_body,
        grid=(grid,),
        in_specs=[
            pl.BlockSpec((_NC, blk, d), lambda i: (0, i, 0)),
            pl.BlockSpec((blk, d), lambda i: (i, 0)),
            pl.BlockSpec((blk, d), lambda i: (i, 0)),
            full(d, d), full(1, d), full(1, d),
            full(d, d), full(1, d), full(1, d),
        ],
        out_specs=pl.BlockSpec((blk, d), lambda i: (i, 0)),
        out_shape=jax.ShapeDtypeStruct((n, d), jnp.float32),
    )(s_parts, base, agts, W_c1, g_n, be_n, W_l, g_l, be_l)


def kernel(agts, ctx, agt_ctrs, ctx_ctrs, W_d0, b_d0, W_d1, g_d, be_d,
           W_q, g_q, be_q, W_c0, g_c0, be_c0, W_c1, W_a, g_n, be_n,
           W_l, g_l, be_l, hi, wi):
    n, d = agts.shape
    row = lambda v: v.reshape(1, d)
    Wc0_d, Wc0_q, Wc0_x = W_c0[:d], W_c0[d:2 * d], W_c0[2 * d:]

    agt_tab, ctx_tab, base = _build_tables(
        agts, ctx, agt_ctrs, ctx_ctrs, W_d0, row(b_d0), W_q, row(g_q),
        row(be_q), Wc0_q, Wc0_x, W_a)

    g1, g2 = _gather_rows(agt_tab, ctx_tab, hi, wi)

    h = _edge_mlp(g1, g2, W_d1, row(g_d), row(be_d), Wc0_d, row(g_c0),
                  row(be_c0))

    s_parts = _scatter_add(h, hi)[:, :n, :]

    return _final(s_parts, base, agts, W_c1, row(g_n), row(be_n), W_l,
                  row(g_l), row(be_l))
